# baseline (speedup 1.0000x reference)
import functools

import jax
import jax.numpy as jnp
from jax import lax
from jax.experimental import pallas as pl
from jax.experimental.pallas import tpu as pltpu
from jax.experimental.pallas import tpu_sc as plsc

NUM_EMBEDDINGS = 1024
EMBED_DIM = 64
BLK = 2048
JT = 128
NT = NUM_EMBEDDINGS // JT


def _l2n(v):
    return v * jax.lax.rsqrt((v * v).sum(axis=-1, keepdims=True) + 1e-12)


def _vq_body(x_ref, cb_ref, proj_ref, disc_ref, idx_ref, cbn_ref,
             cbp_scr, cb2_scr):
    @pl.when(pl.program_id(0) == 0)
    def _():
        cbp = jax.lax.dot_general(
            cb_ref[...], proj_ref[...], (((1,), (0,)), ((), ())),
            preferred_element_type=jnp.float32)
        cbp = _l2n(cbp)
        cbp_scr[...] = cbp
        cb2_scr[...] = (cbp * cbp).sum(axis=1, keepdims=True).reshape(1, -1)
        cbn_ref[:, 0:EMBED_DIM] = _l2n(cb_ref[...])
        cbn_ref[:, EMBED_DIM:2 * EMBED_DIM] = jnp.zeros(
            (NUM_EMBEDDINGS, EMBED_DIM), jnp.float32)

    xp = jax.lax.dot_general(
        x_ref[...], proj_ref[...], (((1,), (0,)), ((), ())),
        preferred_element_type=jnp.float32)
    xp = _l2n(xp)
    x2 = (xp * xp).sum(axis=1, keepdims=True)

    run_min = None
    run_j = None
    lane = jax.lax.broadcasted_iota(jnp.int32, (BLK, JT), 1)
    for t in range(NT):
        dots_t = jax.lax.dot_general(
            xp, cbp_scr[t * JT:(t + 1) * JT, :], (((1,), (1,)), ((), ())),
            preferred_element_type=jnp.float32)
        d_t = (x2 + (-2.0) * dots_t) + cb2_scr[:, t * JT:(t + 1) * JT]
        if t == 0:
            run_min = d_t
            run_j = lane
        else:
            pred = d_t < run_min
            run_min = jnp.where(pred, d_t, run_min)
            run_j = jnp.where(pred, lane + t * JT, run_j)

    m = jnp.min(run_min, axis=1, keepdims=True)
    idx = jnp.min(jnp.where(run_min == m, run_j, NUM_EMBEDDINGS),
                  axis=1, keepdims=True)
    idx_ref[...] = idx

    for t in range(NT):
        disc_ref[:, t * JT:(t + 1) * JT] = (
            lane + t * JT == idx).astype(jnp.float32)


def _sc_gather(cbn, idx_flat):
    info = plsc.get_sparse_core_info()
    nw = info.num_cores * info.num_subcores
    b = idx_flat.shape[0]
    b_per_w = b // nw
    mesh = plsc.VectorSubcoreMesh(core_axis_name="c", subcore_axis_name="s")

    @functools.partial(
        pl.kernel, mesh=mesh,
        out_type=jax.ShapeDtypeStruct((b, 2 * EMBED_DIM), jnp.float32),
        scratch_types=[
            pltpu.VMEM((b_per_w,), jnp.int32),
            pltpu.VMEM((b_per_w, 2 * EMBED_DIM), jnp.float32),
            pltpu.SemaphoreType.DMA,
        ],
    )
    def k(table_hbm, idx_hbm, out_hbm, idx_v, rows_v, sem):
        wid = lax.axis_index("s") * info.num_cores + lax.axis_index("c")
        base = wid * b_per_w
        pltpu.sync_copy(idx_hbm.at[pl.ds(base, b_per_w)], idx_v)
        pltpu.async_copy(table_hbm.at[idx_v], rows_v, sem).wait()
        pltpu.sync_copy(rows_v, out_hbm.at[pl.ds(base, b_per_w)])

    return k(cbn, idx_flat)


def kernel(x, codebook, proj_kernel):
    x_flat = x.reshape(-1, EMBED_DIM)
    n = x_flat.shape[0]
    disc, idx2d, cbn = pl.pallas_call(
        _vq_body,
        grid=(n // BLK,),
        in_specs=[
            pl.BlockSpec((BLK, EMBED_DIM), lambda i: (i, 0)),
            pl.BlockSpec((NUM_EMBEDDINGS, EMBED_DIM), lambda i: (0, 0)),
            pl.BlockSpec((EMBED_DIM, EMBED_DIM), lambda i: (0, 0)),
        ],
        out_specs=[
            pl.BlockSpec((BLK, NUM_EMBEDDINGS), lambda i: (i, 0)),
            pl.BlockSpec((BLK, 1), lambda i: (i, 0)),
            pl.BlockSpec((NUM_EMBEDDINGS, 2 * EMBED_DIM), lambda i: (0, 0)),
        ],
        out_shape=[
            jax.ShapeDtypeStruct((n, NUM_EMBEDDINGS), jnp.float32),
            jax.ShapeDtypeStruct((n, 1), jnp.int32),
            jax.ShapeDtypeStruct((NUM_EMBEDDINGS, 2 * EMBED_DIM), jnp.float32),
        ],
        scratch_shapes=[
            pltpu.VMEM((NUM_EMBEDDINGS, EMBED_DIM), jnp.float32),
            pltpu.VMEM((1, NUM_EMBEDDINGS), jnp.float32),
        ],
    )(x_flat, codebook, proj_kernel)
    quant = _sc_gather(cbn, idx2d.reshape(-1))[:, :EMBED_DIM]
    return disc, quant.reshape(x.shape[:-1] + (EMBED_DIM,))


# speedup vs baseline: 1.2486x; 1.2486x over previous
import jax
import jax.numpy as jnp
from jax.experimental import pallas as pl
from jax.experimental.pallas import tpu as pltpu

NUM_EMBEDDINGS = 1024
EMBED_DIM = 64
BLK = 1024
JT = 128
NT = NUM_EMBEDDINGS // JT


def _l2n(v):
    return v * jax.lax.rsqrt((v * v).sum(axis=-1, keepdims=True) + 1e-12)


def _vq_body(x_ref, cb_ref, proj_ref, disc_ref, quant_ref, cbp_scr, cb2_scr,
             cbn_scr):
    @pl.when(pl.program_id(0) == 0)
    def _():
        cbp = jax.lax.dot_general(
            cb_ref[...], proj_ref[...], (((1,), (0,)), ((), ())),
            preferred_element_type=jnp.float32)
        cbp = _l2n(cbp)
        cbp_scr[...] = cbp
        cb2_scr[...] = (cbp * cbp).sum(axis=1, keepdims=True).reshape(1, -1)
        cbn_scr[...] = _l2n(cb_ref[...])

    xp = jax.lax.dot_general(
        x_ref[...], proj_ref[...], (((1,), (0,)), ((), ())),
        preferred_element_type=jnp.float32)
    xp = _l2n(xp)
    x2 = (xp * xp).sum(axis=1, keepdims=True)

    run_min = None
    run_j = None
    lane = jax.lax.broadcasted_iota(jnp.int32, (BLK, JT), 1)
    for t in range(NT):
        dots_t = jax.lax.dot_general(
            xp, cbp_scr[t * JT:(t + 1) * JT, :], (((1,), (1,)), ((), ())),
            preferred_element_type=jnp.float32)
        d_t = (x2 + (-2.0) * dots_t) + cb2_scr[:, t * JT:(t + 1) * JT]
        if t == 0:
            run_min = d_t
            run_j = lane
        else:
            pred = d_t < run_min
            run_min = jnp.where(pred, d_t, run_min)
            run_j = jnp.where(pred, lane + t * JT, run_j)

    m = jnp.min(run_min, axis=1, keepdims=True)
    idx = jnp.min(jnp.where(run_min == m, run_j, NUM_EMBEDDINGS),
                  axis=1, keepdims=True)

    q = jnp.zeros((BLK, EMBED_DIM), jnp.float32)
    for t in range(NT):
        disc_t = (lane + t * JT == idx).astype(jnp.float32)
        disc_ref[:, t * JT:(t + 1) * JT] = disc_t
        q = q + jax.lax.dot_general(
            disc_t, cbn_scr[t * JT:(t + 1) * JT, :], (((1,), (0,)), ((), ())),
            preferred_element_type=jnp.float32)
    quant_ref[...] = q


def kernel(x, codebook, proj_kernel):
    x_flat = x.reshape(-1, EMBED_DIM)
    n = x_flat.shape[0]
    grid = n // BLK
    disc, quant = pl.pallas_call(
        _vq_body,
        grid=(grid,),
        in_specs=[
            pl.BlockSpec((BLK, EMBED_DIM), lambda i: (i, 0)),
            pl.BlockSpec((NUM_EMBEDDINGS, EMBED_DIM), lambda i: (0, 0)),
            pl.BlockSpec((EMBED_DIM, EMBED_DIM), lambda i: (0, 0)),
        ],
        out_specs=[
            pl.BlockSpec((BLK, NUM_EMBEDDINGS), lambda i: (i, 0)),
            pl.BlockSpec((BLK, EMBED_DIM), lambda i: (i, 0)),
        ],
        out_shape=[
            jax.ShapeDtypeStruct((n, NUM_EMBEDDINGS), jnp.float32),
            jax.ShapeDtypeStruct((n, EMBED_DIM), jnp.float32),
        ],
        scratch_shapes=[
            pltpu.VMEM((NUM_EMBEDDINGS, EMBED_DIM), jnp.float32),
            pltpu.VMEM((1, NUM_EMBEDDINGS), jnp.float32),
            pltpu.VMEM((NUM_EMBEDDINGS, EMBED_DIM), jnp.float32),
        ],
    )(x_flat, codebook, proj_kernel)
    return disc, quant.reshape(x.shape[:-1] + (EMBED_DIM,))


# speedup vs baseline: 1.3494x; 1.0808x over previous
import jax
import jax.numpy as jnp
from jax.experimental import pallas as pl
from jax.experimental.pallas import tpu as pltpu

NUM_EMBEDDINGS = 1024
EMBED_DIM = 64
BLK = 2048
RC = 512
JT = 128
NT = NUM_EMBEDDINGS // JT


def _l2n(v):
    return v * jax.lax.rsqrt((v * v).sum(axis=-1, keepdims=True) + 1e-12)


def _vq_body(x_ref, cb_ref, proj_ref, disc_ref, quant_ref, cbp_scr, cb2_scr,
             cbn_scr):
    @pl.when(pl.program_id(0) == 0)
    def _():
        cbp = jax.lax.dot_general(
            cb_ref[...], proj_ref[...], (((1,), (0,)), ((), ())),
            preferred_element_type=jnp.float32)
        cbp = _l2n(cbp)
        cbp_scr[...] = cbp
        cb2_scr[...] = (cbp * cbp).sum(axis=1, keepdims=True).reshape(1, -1)
        cbn_scr[...] = _l2n(cb_ref[...])

    lane = jax.lax.broadcasted_iota(jnp.int32, (RC, JT), 1)
    for rc in range(BLK // RC):
        r0 = rc * RC
        xp = jax.lax.dot_general(
            x_ref[r0:r0 + RC, :], proj_ref[...], (((1,), (0,)), ((), ())),
            preferred_element_type=jnp.float32)
        xp = _l2n(xp)
        x2 = (xp * xp).sum(axis=1, keepdims=True)

        run_min = None
        run_j = None
        for t in range(NT):
            dots_t = jax.lax.dot_general(
                xp, cbp_scr[t * JT:(t + 1) * JT, :], (((1,), (1,)), ((), ())),
                preferred_element_type=jnp.float32)
            d_t = (x2 + (-2.0) * dots_t) + cb2_scr[:, t * JT:(t + 1) * JT]
            if t == 0:
                run_min = d_t
                run_j = lane
            else:
                pred = d_t < run_min
                run_min = jnp.where(pred, d_t, run_min)
                run_j = jnp.where(pred, lane + t * JT, run_j)

        m = jnp.min(run_min, axis=1, keepdims=True)
        idx = jnp.min(jnp.where(run_min == m, run_j, NUM_EMBEDDINGS),
                      axis=1, keepdims=True)

        q = jnp.zeros((RC, EMBED_DIM), jnp.float32)
        for t in range(NT):
            disc_t = (lane + t * JT == idx).astype(jnp.float32)
            disc_ref[r0:r0 + RC, t * JT:(t + 1) * JT] = disc_t
            q = q + jax.lax.dot_general(
                disc_t, cbn_scr[t * JT:(t + 1) * JT, :],
                (((1,), (0,)), ((), ())),
                preferred_element_type=jnp.float32)
        quant_ref[r0:r0 + RC, :] = q


def kernel(x, codebook, proj_kernel):
    x_flat = x.reshape(-1, EMBED_DIM)
    n = x_flat.shape[0]
    grid = n // BLK
    disc, quant = pl.pallas_call(
        _vq_body,
        grid=(grid,),
        in_specs=[
            pl.BlockSpec((BLK, EMBED_DIM), lambda i: (i, 0)),
            pl.BlockSpec((NUM_EMBEDDINGS, EMBED_DIM), lambda i: (0, 0)),
            pl.BlockSpec((EMBED_DIM, EMBED_DIM), lambda i: (0, 0)),
        ],
        out_specs=[
            pl.BlockSpec((BLK, NUM_EMBEDDINGS), lambda i: (i, 0)),
            pl.BlockSpec((BLK, EMBED_DIM), lambda i: (i, 0)),
        ],
        out_shape=[
            jax.ShapeDtypeStruct((n, NUM_EMBEDDINGS), jnp.float32),
            jax.ShapeDtypeStruct((n, EMBED_DIM), jnp.float32),
        ],
        scratch_shapes=[
            pltpu.VMEM((NUM_EMBEDDINGS, EMBED_DIM), jnp.float32),
            pltpu.VMEM((1, NUM_EMBEDDINGS), jnp.float32),
            pltpu.VMEM((NUM_EMBEDDINGS, EMBED_DIM), jnp.float32),
        ],
    )(x_flat, codebook, proj_kernel)
    return disc, quant.reshape(x.shape[:-1] + (EMBED_DIM,))
